# baseline (speedup 1.0000x reference)
import functools

import jax
import jax.numpy as jnp
from jax import lax
from jax.experimental import pallas as pl
from jax.experimental.pallas import tpu as pltpu
from jax.experimental.pallas import tpu_sc as plsc

_N = 10000
_E = 160000
_IN = 256
_OUT = 256
_HID = 512

_NPAD = 10240
_EPAD = 163840
_NC = 2
_NS = 16
_F = 128
_CHUNK = 128


def _vmesh():
    return plsc.VectorSubcoreMesh(core_axis_name="c", subcore_axis_name="s")


def _sc_params():
    return pltpu.CompilerParams(needs_layout_passes=False)


def _deg_call(dst2d):
    per_w = _EPAD // (_NC * _NS)
    n_chunks = per_w // _CHUNK
    wb = _NPAD // _NS

    @functools.partial(
        pl.kernel,
        out_type=jax.ShapeDtypeStruct((_NC * _NPAD, _F), jnp.float32),
        mesh=_vmesh(),
        scratch_types=[
            pltpu.VMEM((n_chunks, _CHUNK), jnp.int32),
            pltpu.VMEM((_CHUNK, _F), jnp.float32),
            pltpu.VMEM((_CHUNK, _F), jnp.float32),
            pltpu.VMEM_SHARED((_NPAD, _F), jnp.float32),
        ],
        compiler_params=_sc_params(),
    )
    def k(dst_hbm, out_hbm, didx_v, ones_v, zbuf_v, hist_sh):
        c = lax.axis_index("c")
        s = lax.axis_index("s")
        w = c * _NS + s

        pltpu.sync_copy(dst_hbm.at[pl.ds(w * n_chunks, n_chunks)], didx_v)

        @pl.loop(0, _CHUNK)
        def _(i):
            for g in range(_F // 16):
                ones_v[i, pl.ds(g * 16, 16)] = jnp.full((16,), 1.0,
                                                        jnp.float32)
                zbuf_v[i, pl.ds(g * 16, 16)] = jnp.zeros((16,), jnp.float32)

        for kk in range(wb // _CHUNK):
            pltpu.sync_copy(zbuf_v,
                            hist_sh.at[pl.ds(s * wb + kk * _CHUNK, _CHUNK)])
        plsc.subcore_barrier()

        @pl.loop(0, n_chunks)
        def _(t):
            pltpu.sync_copy(ones_v, hist_sh.at[didx_v.at[t]], add=True)

        plsc.subcore_barrier()
        pltpu.sync_copy(hist_sh.at[pl.ds(s * wb, wb)],
                        out_hbm.at[pl.ds(c * _NPAD + s * wb, wb)])

    return k(dst2d)


_Q = _NPAD // 4
_CAP = 6144


def _agg_call(xs, src2d, dst2d):
    per_s = _EPAD // _NS
    n_chunks = per_s // _CHUNK
    nstage = n_chunks // 2
    wq = _Q // _NS

    @functools.partial(
        pl.kernel,
        out_type=jax.ShapeDtypeStruct((_NPAD, 2, _F), jnp.float32),
        mesh=_vmesh(),
        scratch_types=[
            pltpu.VMEM((nstage, _CHUNK), jnp.int32),
            pltpu.VMEM((nstage, _CHUNK), jnp.int32),
            pltpu.VMEM((_CAP // _CHUNK, _CHUNK), jnp.int32),
            pltpu.VMEM((_CAP // _CHUNK, _CHUNK), jnp.int32),
            pltpu.VMEM((_CHUNK, 2, _F), jnp.float32),
            pltpu.VMEM((_CHUNK, 2, _F), jnp.float32),
            pltpu.VMEM_SHARED((_Q, 2, _F), jnp.float32),
            pltpu.SemaphoreType.DMA,
            pltpu.SemaphoreType.DMA,
        ],
        compiler_params=_sc_params(),
    )
    def k(xs_hbm, src_hbm, dst_hbm, out_hbm,
          sraw_v, draw_v, csrc_v, cdst_v, rows0_v, rows1_v,
          acc_sh, sem0, sem1):
        c = lax.axis_index("c")
        s = lax.axis_index("s")

        for p in range(2):
            q = c * 2 + p
            lo = q * _Q
            hi = lo + _Q

            @pl.loop(0, _CHUNK)
            def _(i):
                for h in range(2):
                    for g in range(_F // 16):
                        rows0_v[i, h, pl.ds(g * 16, 16)] = jnp.zeros(
                            (16,), jnp.float32)

            pltpu.sync_copy(rows0_v, acc_sh.at[pl.ds(s * wq, _CHUNK)])
            pltpu.sync_copy(rows0_v.at[pl.ds(0, wq - _CHUNK)],
                            acc_sh.at[pl.ds(s * wq + _CHUNK, wq - _CHUNK)])

            @pl.loop(0, _CAP // _CHUNK)
            def _(i):
                for g in range(_CHUNK // 16):
                    csrc_v[i, pl.ds(g * 16, 16)] = jnp.full((16,), _N,
                                                            jnp.int32)
                    cdst_v[i, pl.ds(g * 16, 16)] = jnp.zeros((16,),
                                                             jnp.int32)

            cnt = jnp.int32(0)
            for phase in range(2):
                pbase = s * n_chunks + phase * nstage
                pltpu.sync_copy(src_hbm.at[pl.ds(pbase, nstage)], sraw_v)
                pltpu.sync_copy(dst_hbm.at[pl.ds(pbase, nstage)], draw_v)

                def rowbody(j, cnt):
                    for g in range(_CHUNK // 16):
                        d = draw_v[j, pl.ds(g * 16, 16)]
                        sv = sraw_v[j, pl.ds(g * 16, 16)]
                        mask = (d >= lo) & (d < hi)
                        mi = mask.astype(jnp.int32)
                        pos = jnp.minimum(cnt + plsc.cumsum(mi) - 1,
                                          _CAP - 1)
                        prow = jnp.right_shift(pos, 7)
                        plane = jnp.bitwise_and(pos, _CHUNK - 1)
                        plsc.store_scatter(cdst_v, [prow, plane], d - lo,
                                           mask=mask)
                        plsc.store_scatter(csrc_v, [prow, plane], sv,
                                           mask=mask)
                        cnt = cnt + lax.reduce_sum(mi, axes=(0,))
                    return cnt

                cnt = lax.fori_loop(0, nstage, rowbody, cnt)

            nu = (cnt + 2 * _CHUNK - 1) // (2 * _CHUNK)
            ncl = 2 * nu
            plsc.subcore_barrier()

            pltpu.async_copy(xs_hbm.at[csrc_v.at[jnp.int32(0)]],
                             rows0_v, sem0)

            @pl.loop(0, nu)
            def _(u):
                j0 = 2 * u
                j1 = j0 + 1

                pltpu.async_copy(xs_hbm.at[csrc_v.at[j1]], rows1_v, sem1)
                pltpu.make_async_copy(xs_hbm.at[csrc_v.at[j0]],
                                      rows0_v, sem0).wait()
                pltpu.sync_copy(rows0_v, acc_sh.at[cdst_v.at[j0]], add=True)

                @pl.when(j0 + 2 < ncl)
                def _():
                    pltpu.async_copy(xs_hbm.at[csrc_v.at[j0 + 2]],
                                     rows0_v, sem0)

                pltpu.make_async_copy(xs_hbm.at[csrc_v.at[j1]],
                                      rows1_v, sem1).wait()
                pltpu.sync_copy(rows1_v, acc_sh.at[cdst_v.at[j1]],
                                add=True)

            plsc.subcore_barrier()
            pltpu.sync_copy(acc_sh.at[pl.ds(s * wq, wq)],
                            out_hbm.at[pl.ds(lo + s * wq, wq)])

    return k(xs, src2d, dst2d)


_R = 1024


def _dense_call(agg1, x_pad, dis_c, inv_c, W1, b1r, W2):
    def body(agg_ref, x_ref, dis_ref, inv_ref, w1_ref, b1_ref, w2_ref,
             xs2_ref, p_ref):
        dis = dis_ref[...]
        inv = inv_ref[...]
        z1 = dis * agg_ref[...] + inv * x_ref[...]
        h1 = jnp.maximum(
            jnp.dot(z1, w1_ref[...], preferred_element_type=jnp.float32)
            + b1_ref[...], 0.0)
        h2 = jnp.dot(h1, w2_ref[...], preferred_element_type=jnp.float32)
        xs2_ref[...] = dis * h2
        p_ref[...] = inv * h2

    return pl.pallas_call(
        body,
        grid=(_NPAD // _R,),
        in_specs=[
            pl.BlockSpec((_R, _IN), lambda i: (i, 0)),
            pl.BlockSpec((_R, _IN), lambda i: (i, 0)),
            pl.BlockSpec((_R, 1), lambda i: (i, 0)),
            pl.BlockSpec((_R, 1), lambda i: (i, 0)),
            pl.BlockSpec((_IN, _HID), lambda i: (0, 0)),
            pl.BlockSpec((1, _HID), lambda i: (0, 0)),
            pl.BlockSpec((_HID, _OUT), lambda i: (0, 0)),
        ],
        out_specs=[
            pl.BlockSpec((_R, _OUT), lambda i: (i, 0)),
            pl.BlockSpec((_R, _OUT), lambda i: (i, 0)),
        ],
        out_shape=[
            jax.ShapeDtypeStruct((_NPAD, _OUT), jnp.float32),
            jax.ShapeDtypeStruct((_NPAD, _OUT), jnp.float32),
        ],
    )(agg1, x_pad, dis_c, inv_c, W1, b1r, W2)


def _final_call(agg2, p, dis_c, b2r):
    def body(agg_ref, p_ref, dis_ref, b2_ref, o_ref):
        o_ref[...] = jnp.maximum(
            dis_ref[...] * agg_ref[...] + p_ref[...] + b2_ref[...], 0.0)

    return pl.pallas_call(
        body,
        grid=(_NPAD // _R,),
        in_specs=[
            pl.BlockSpec((_R, _OUT), lambda i: (i, 0)),
            pl.BlockSpec((_R, _OUT), lambda i: (i, 0)),
            pl.BlockSpec((_R, 1), lambda i: (i, 0)),
            pl.BlockSpec((1, _OUT), lambda i: (0, 0)),
        ],
        out_specs=pl.BlockSpec((_R, _OUT), lambda i: (i, 0)),
        out_shape=jax.ShapeDtypeStruct((_NPAD, _OUT), jnp.float32),
    )(agg2, p, dis_c, b2r)


def kernel(x, edge_index, W1, b1, W2, b2):
    src = edge_index[0]
    dst = edge_index[1]
    npad_e = _EPAD - _E
    src_pad = jnp.concatenate(
        [src, jnp.full((npad_e,), _N, jnp.int32)])
    dst_pad = jnp.concatenate(
        [dst, jnp.full((npad_e,), _NPAD - 1, jnp.int32)])
    x_pad = jnp.concatenate(
        [x, jnp.zeros((_NPAD - _N, _IN), x.dtype)], axis=0)
    src2d = src_pad.reshape(_EPAD // _CHUNK, _CHUNK)
    dst2d = dst_pad.reshape(_EPAD // _CHUNK, _CHUNK)

    deg_parts = _deg_call(dst2d)
    deg = deg_parts[:_NPAD, 0] + deg_parts[_NPAD:, 0] + 1.0
    dis = jnp.where(jnp.arange(_NPAD) < _N, lax.rsqrt(deg), 0.0)
    inv = 1.0 / deg
    dis_c = dis[:, None]
    inv_c = inv[:, None]

    xs = dis_c * x_pad

    agg1 = _agg_call(xs.reshape(_NPAD, 2, _F), src2d, dst2d
                     ).reshape(_NPAD, _IN)
    xs2, p = _dense_call(agg1, x_pad, dis_c, inv_c,
                         W1, b1.reshape(1, -1), W2)
    agg2 = _agg_call(xs2.reshape(_NPAD, 2, _F), src2d, dst2d
                     ).reshape(_NPAD, _OUT)
    out = _final_call(agg2, p, dis_c, b2.reshape(1, -1))
    return out[:_N]


# speedup vs baseline: 3.8368x; 3.8368x over previous
import functools

import jax
import jax.numpy as jnp
from jax import lax
from jax.experimental import pallas as pl
from jax.experimental.pallas import tpu as pltpu
from jax.experimental.pallas import tpu_sc as plsc

_N = 10000
_E = 160000
_IN = 256
_OUT = 256
_HID = 512

_NPAD = 10240
_EPAD = 163840
_NC = 2
_NS = 16
_F = 128
_CHUNK = 128


def _vmesh():
    return plsc.VectorSubcoreMesh(core_axis_name="c", subcore_axis_name="s")


def _sc_params():
    return pltpu.CompilerParams(needs_layout_passes=False)


def _deg_call(dst2d):
    per_w = _EPAD // (_NC * _NS)
    n_chunks = per_w // _CHUNK
    wb = _NPAD // _NS

    @functools.partial(
        pl.kernel,
        out_type=jax.ShapeDtypeStruct((_NC * _NPAD, _F), jnp.float32),
        mesh=_vmesh(),
        scratch_types=[
            pltpu.VMEM((n_chunks, _CHUNK), jnp.int32),
            pltpu.VMEM((_CHUNK, _F), jnp.float32),
            pltpu.VMEM((_CHUNK, _F), jnp.float32),
            pltpu.VMEM_SHARED((_NPAD, _F), jnp.float32),
        ],
        compiler_params=_sc_params(),
    )
    def k(dst_hbm, out_hbm, didx_v, ones_v, zbuf_v, hist_sh):
        c = lax.axis_index("c")
        s = lax.axis_index("s")
        w = c * _NS + s

        pltpu.sync_copy(dst_hbm.at[pl.ds(w * n_chunks, n_chunks)], didx_v)

        @pl.loop(0, _CHUNK)
        def _(i):
            for g in range(_F // 16):
                ones_v[i, pl.ds(g * 16, 16)] = jnp.full((16,), 1.0,
                                                        jnp.float32)
                zbuf_v[i, pl.ds(g * 16, 16)] = jnp.zeros((16,), jnp.float32)

        for kk in range(wb // _CHUNK):
            pltpu.sync_copy(zbuf_v,
                            hist_sh.at[pl.ds(s * wb + kk * _CHUNK, _CHUNK)])
        plsc.subcore_barrier()

        @pl.loop(0, n_chunks)
        def _(t):
            pltpu.sync_copy(ones_v, hist_sh.at[didx_v.at[t]], add=True)

        plsc.subcore_barrier()
        pltpu.sync_copy(hist_sh.at[pl.ds(s * wb, wb)],
                        out_hbm.at[pl.ds(c * _NPAD + s * wb, wb)])

    return k(dst2d)


_Q = _NPAD // 4
_CAP = 6144


def _agg_call(xs, src2d, dst2d):
    per_s = _EPAD // _NS
    n_chunks = per_s // _CHUNK
    nstage = n_chunks // 2
    wq = _Q // _NS

    @functools.partial(
        pl.kernel,
        out_type=jax.ShapeDtypeStruct((_NPAD, 2, _F), jnp.float32),
        mesh=_vmesh(),
        scratch_types=[
            pltpu.VMEM((nstage, _CHUNK), jnp.int32),
            pltpu.VMEM((nstage, _CHUNK), jnp.int32),
            pltpu.VMEM((_CAP // _CHUNK, _CHUNK), jnp.int32),
            pltpu.VMEM((_CAP // _CHUNK, _CHUNK), jnp.int32),
            pltpu.VMEM((_CHUNK, 2, _F), jnp.float32),
            pltpu.VMEM((_CHUNK, 2, _F), jnp.float32),
            pltpu.VMEM_SHARED((_Q, 2, _F), jnp.float32),
            pltpu.SemaphoreType.DMA,
            pltpu.SemaphoreType.DMA,
        ],
        compiler_params=_sc_params(),
    )
    def k(xs_hbm, src_hbm, dst_hbm, out_hbm,
          sraw_v, draw_v, csrc_v, cdst_v, rows0_v, rows1_v,
          acc_sh, sem0, sem1):
        c = lax.axis_index("c")
        s = lax.axis_index("s")

        for p in range(2):
            q = c * 2 + p
            lo = q * _Q
            hi = lo + _Q

            @pl.loop(0, _CHUNK)
            def _(i):
                for h in range(2):
                    for g in range(_F // 16):
                        rows0_v[i, h, pl.ds(g * 16, 16)] = jnp.zeros(
                            (16,), jnp.float32)

            pltpu.sync_copy(rows0_v, acc_sh.at[pl.ds(s * wq, _CHUNK)])
            pltpu.sync_copy(rows0_v.at[pl.ds(0, wq - _CHUNK)],
                            acc_sh.at[pl.ds(s * wq + _CHUNK, wq - _CHUNK)])

            @pl.loop(0, _CAP // _CHUNK)
            def _(i):
                for g in range(_CHUNK // 16):
                    flat = i * _CHUNK + g * 16 + lax.iota(jnp.int32, 16)
                    csrc_v[i, pl.ds(g * 16, 16)] = (
                        _N + jnp.remainder(flat, _NPAD - _N))
                    cdst_v[i, pl.ds(g * 16, 16)] = jnp.remainder(flat, _Q)

            cnt = jnp.int32(0)
            for phase in range(2):
                pbase = s * n_chunks + phase * nstage
                pltpu.sync_copy(src_hbm.at[pl.ds(pbase, nstage)], sraw_v)
                pltpu.sync_copy(dst_hbm.at[pl.ds(pbase, nstage)], draw_v)

                def rowbody(j, cnt):
                    for g in range(_CHUNK // 16):
                        d = draw_v[j, pl.ds(g * 16, 16)]
                        sv = sraw_v[j, pl.ds(g * 16, 16)]
                        mask = (d >= lo) & (d < hi)
                        mi = mask.astype(jnp.int32)
                        pos = jnp.minimum(cnt + plsc.cumsum(mi) - 1,
                                          _CAP - 1)
                        prow = jnp.right_shift(pos, 7)
                        plane = jnp.bitwise_and(pos, _CHUNK - 1)
                        plsc.store_scatter(cdst_v, [prow, plane], d - lo,
                                           mask=mask)
                        plsc.store_scatter(csrc_v, [prow, plane], sv,
                                           mask=mask)
                        cnt = cnt + lax.reduce_sum(mi, axes=(0,))
                    return cnt

                cnt = lax.fori_loop(0, nstage, rowbody, cnt)

            nu = (cnt + 2 * _CHUNK - 1) // (2 * _CHUNK)
            ncl = 2 * nu
            plsc.subcore_barrier()

            pltpu.async_copy(xs_hbm.at[csrc_v.at[jnp.int32(0)]],
                             rows0_v, sem0)

            @pl.loop(0, nu)
            def _(u):
                j0 = 2 * u
                j1 = j0 + 1

                pltpu.async_copy(xs_hbm.at[csrc_v.at[j1]], rows1_v, sem1)
                pltpu.make_async_copy(xs_hbm.at[csrc_v.at[j0]],
                                      rows0_v, sem0).wait()
                pltpu.sync_copy(rows0_v, acc_sh.at[cdst_v.at[j0]], add=True)

                @pl.when(j0 + 2 < ncl)
                def _():
                    pltpu.async_copy(xs_hbm.at[csrc_v.at[j0 + 2]],
                                     rows0_v, sem0)

                pltpu.make_async_copy(xs_hbm.at[csrc_v.at[j1]],
                                      rows1_v, sem1).wait()
                pltpu.sync_copy(rows1_v, acc_sh.at[cdst_v.at[j1]],
                                add=True)

            plsc.subcore_barrier()
            pltpu.sync_copy(acc_sh.at[pl.ds(s * wq, wq)],
                            out_hbm.at[pl.ds(lo + s * wq, wq)])

    return k(xs, src2d, dst2d)


_R = 1024


def _dense_call(agg1, x_pad, dis_c, inv_c, W1, b1r, W2):
    def body(agg_ref, x_ref, dis_ref, inv_ref, w1_ref, b1_ref, w2_ref,
             xs2_ref, p_ref):
        dis = dis_ref[...]
        inv = inv_ref[...]
        z1 = dis * agg_ref[...] + inv * x_ref[...]
        h1 = jnp.maximum(
            jnp.dot(z1, w1_ref[...], preferred_element_type=jnp.float32)
            + b1_ref[...], 0.0)
        h2 = jnp.dot(h1, w2_ref[...], preferred_element_type=jnp.float32)
        xs2_ref[...] = dis * h2
        p_ref[...] = inv * h2

    return pl.pallas_call(
        body,
        grid=(_NPAD // _R,),
        in_specs=[
            pl.BlockSpec((_R, _IN), lambda i: (i, 0)),
            pl.BlockSpec((_R, _IN), lambda i: (i, 0)),
            pl.BlockSpec((_R, 1), lambda i: (i, 0)),
            pl.BlockSpec((_R, 1), lambda i: (i, 0)),
            pl.BlockSpec((_IN, _HID), lambda i: (0, 0)),
            pl.BlockSpec((1, _HID), lambda i: (0, 0)),
            pl.BlockSpec((_HID, _OUT), lambda i: (0, 0)),
        ],
        out_specs=[
            pl.BlockSpec((_R, _OUT), lambda i: (i, 0)),
            pl.BlockSpec((_R, _OUT), lambda i: (i, 0)),
        ],
        out_shape=[
            jax.ShapeDtypeStruct((_NPAD, _OUT), jnp.float32),
            jax.ShapeDtypeStruct((_NPAD, _OUT), jnp.float32),
        ],
    )(agg1, x_pad, dis_c, inv_c, W1, b1r, W2)


def _final_call(agg2, p, dis_c, b2r):
    def body(agg_ref, p_ref, dis_ref, b2_ref, o_ref):
        o_ref[...] = jnp.maximum(
            dis_ref[...] * agg_ref[...] + p_ref[...] + b2_ref[...], 0.0)

    return pl.pallas_call(
        body,
        grid=(_NPAD // _R,),
        in_specs=[
            pl.BlockSpec((_R, _OUT), lambda i: (i, 0)),
            pl.BlockSpec((_R, _OUT), lambda i: (i, 0)),
            pl.BlockSpec((_R, 1), lambda i: (i, 0)),
            pl.BlockSpec((1, _OUT), lambda i: (0, 0)),
        ],
        out_specs=pl.BlockSpec((_R, _OUT), lambda i: (i, 0)),
        out_shape=jax.ShapeDtypeStruct((_NPAD, _OUT), jnp.float32),
    )(agg2, p, dis_c, b2r)


def kernel(x, edge_index, W1, b1, W2, b2):
    src = edge_index[0]
    dst = edge_index[1]
    n_grp = _EPAD // 5120
    per_grp_pad = 5120 - _E // n_grp
    pad_rows = _N + (jnp.arange(per_grp_pad, dtype=jnp.int32) * 2
                     ) % (_NPAD - _N)
    pad_blk = jnp.broadcast_to(pad_rows, (n_grp, per_grp_pad))
    src_pad = jnp.concatenate(
        [src.reshape(n_grp, -1), pad_blk], axis=1).reshape(-1)
    dst_pad = jnp.concatenate(
        [dst.reshape(n_grp, -1), pad_blk], axis=1).reshape(-1)
    x_pad = jnp.concatenate(
        [x, jnp.zeros((_NPAD - _N, _IN), x.dtype)], axis=0)
    src2d = src_pad.reshape(_EPAD // _CHUNK, _CHUNK)
    dst2d = dst_pad.reshape(_EPAD // _CHUNK, _CHUNK)

    deg_parts = _deg_call(dst2d)
    deg = deg_parts[:_NPAD, 0] + deg_parts[_NPAD:, 0] + 1.0
    dis = jnp.where(jnp.arange(_NPAD) < _N, lax.rsqrt(deg), 0.0)
    inv = 1.0 / deg
    dis_c = dis[:, None]
    inv_c = inv[:, None]

    xs = dis_c * x_pad

    agg1 = _agg_call(xs.reshape(_NPAD, 2, _F), src2d, dst2d
                     ).reshape(_NPAD, _IN)
    xs2, p = _dense_call(agg1, x_pad, dis_c, inv_c,
                         W1, b1.reshape(1, -1), W2)
    agg2 = _agg_call(xs2.reshape(_NPAD, 2, _F), src2d, dst2d
                     ).reshape(_NPAD, _OUT)
    out = _final_call(agg2, p, dis_c, b2.reshape(1, -1))
    return out[:_N]


# speedup vs baseline: 3.8368x; 1.0000x over previous
import functools

import jax
import jax.numpy as jnp
from jax import lax
from jax.experimental import pallas as pl
from jax.experimental.pallas import tpu as pltpu
from jax.experimental.pallas import tpu_sc as plsc

_N = 10000
_E = 160000
_IN = 256
_OUT = 256
_HID = 512

_NPAD = 10240
_EPAD = 163840
_NC = 2
_NS = 16
_F = 128
_CHUNK = 128


def _vmesh():
    return plsc.VectorSubcoreMesh(core_axis_name="c", subcore_axis_name="s")


def _sc_params():
    return pltpu.CompilerParams(needs_layout_passes=False)


def _deg_call(dst2d):
    per_w = _EPAD // (_NC * _NS)
    n_chunks = per_w // _CHUNK
    wb = _NPAD // _NS

    @functools.partial(
        pl.kernel,
        out_type=jax.ShapeDtypeStruct((_NC * _NPAD, _F), jnp.float32),
        mesh=_vmesh(),
        scratch_types=[
            pltpu.VMEM((n_chunks, _CHUNK), jnp.int32),
            pltpu.VMEM((_CHUNK, _F), jnp.float32),
            pltpu.VMEM((_CHUNK, _F), jnp.float32),
            pltpu.VMEM_SHARED((_NPAD, _F), jnp.float32),
        ],
        compiler_params=_sc_params(),
    )
    def k(dst_hbm, out_hbm, didx_v, ones_v, zbuf_v, hist_sh):
        c = lax.axis_index("c")
        s = lax.axis_index("s")
        w = c * _NS + s

        pltpu.sync_copy(dst_hbm.at[pl.ds(w * n_chunks, n_chunks)], didx_v)

        @pl.loop(0, _CHUNK)
        def _(i):
            for g in range(_F // 16):
                ones_v[i, pl.ds(g * 16, 16)] = jnp.full((16,), 1.0,
                                                        jnp.float32)
                zbuf_v[i, pl.ds(g * 16, 16)] = jnp.zeros((16,), jnp.float32)

        for kk in range(wb // _CHUNK):
            pltpu.sync_copy(zbuf_v,
                            hist_sh.at[pl.ds(s * wb + kk * _CHUNK, _CHUNK)])
        plsc.subcore_barrier()

        @pl.loop(0, n_chunks)
        def _(t):
            pltpu.sync_copy(ones_v, hist_sh.at[didx_v.at[t]], add=True)

        plsc.subcore_barrier()
        pltpu.sync_copy(hist_sh.at[pl.ds(s * wb, wb)],
                        out_hbm.at[pl.ds(c * _NPAD + s * wb, wb)])

    return k(dst2d)


_Q = _NPAD // 4
_CAP = 6144


def _agg_call(xs, src2d, dst2d):
    per_s = _EPAD // _NS
    n_chunks = per_s // _CHUNK
    nstage = n_chunks // 2
    wq = _Q // _NS

    @functools.partial(
        pl.kernel,
        out_type=jax.ShapeDtypeStruct((_NPAD, 2, _F), jnp.float32),
        mesh=_vmesh(),
        scratch_types=[
            pltpu.VMEM((nstage, _CHUNK), jnp.int32),
            pltpu.VMEM((nstage, _CHUNK), jnp.int32),
            pltpu.VMEM((_CAP // _CHUNK, _CHUNK), jnp.int32),
            pltpu.VMEM((_CAP // _CHUNK, _CHUNK), jnp.int32),
            pltpu.VMEM((_CHUNK, 2, _F), jnp.float32),
            pltpu.VMEM((_CHUNK, 2, _F), jnp.float32),
            pltpu.VMEM_SHARED((_Q, 2, _F), jnp.float32),
            pltpu.SemaphoreType.DMA,
            pltpu.SemaphoreType.DMA,
        ],
        compiler_params=_sc_params(),
    )
    def k(xs_hbm, src_hbm, dst_hbm, out_hbm,
          sraw_v, draw_v, csrc_v, cdst_v, rows0_v, rows1_v,
          acc_sh, sem0, sem1):
        c = lax.axis_index("c")
        s = lax.axis_index("s")

        for p in range(2):
            q = c * 2 + p
            lo = q * _Q
            hi = lo + _Q

            @pl.loop(0, _CHUNK)
            def _(i):
                for h in range(2):
                    for g in range(_F // 16):
                        rows0_v[i, h, pl.ds(g * 16, 16)] = jnp.zeros(
                            (16,), jnp.float32)

            pltpu.sync_copy(rows0_v, acc_sh.at[pl.ds(s * wq, _CHUNK)])
            pltpu.sync_copy(rows0_v.at[pl.ds(0, wq - _CHUNK)],
                            acc_sh.at[pl.ds(s * wq + _CHUNK, wq - _CHUNK)])

            @pl.loop(0, _CAP // _CHUNK)
            def _(i):
                for g in range(_CHUNK // 16):
                    flat = i * _CHUNK + g * 16 + lax.iota(jnp.int32, 16)
                    csrc_v[i, pl.ds(g * 16, 16)] = (
                        _N + jnp.remainder(flat, _NPAD - _N))
                    cdst_v[i, pl.ds(g * 16, 16)] = jnp.remainder(flat, _Q)

            cnt = jnp.int32(0)
            for phase in range(2):
                pbase = s * n_chunks + phase * nstage
                pltpu.sync_copy(src_hbm.at[pl.ds(pbase, nstage)], sraw_v)
                pltpu.sync_copy(dst_hbm.at[pl.ds(pbase, nstage)], draw_v)

                def rowbody(j, cnt):
                    for g in range(_CHUNK // 16):
                        d = draw_v[j, pl.ds(g * 16, 16)]
                        sv = sraw_v[j, pl.ds(g * 16, 16)]
                        mask = (d >= lo) & (d < hi)
                        mi = mask.astype(jnp.int32)
                        pos = jnp.minimum(cnt + plsc.cumsum(mi) - 1,
                                          _CAP - 1)
                        prow = jnp.right_shift(pos, 7)
                        plane = jnp.bitwise_and(pos, _CHUNK - 1)
                        plsc.store_scatter(cdst_v, [prow, plane], d - lo,
                                           mask=mask)
                        plsc.store_scatter(csrc_v, [prow, plane], sv,
                                           mask=mask)
                        cnt = cnt + lax.reduce_sum(mi, axes=(0,))
                    return cnt

                cnt = lax.fori_loop(0, nstage, rowbody, cnt)

            nu = (cnt + 2 * _CHUNK - 1) // (2 * _CHUNK)
            ncl = 2 * nu
            plsc.subcore_barrier()

            pltpu.async_copy(xs_hbm.at[csrc_v.at[jnp.int32(0)]],
                             rows0_v, sem0)

            @pl.loop(0, nu)
            def _(u):
                j0 = 2 * u
                j1 = j0 + 1

                pltpu.async_copy(xs_hbm.at[csrc_v.at[j1]], rows1_v, sem1)
                pltpu.make_async_copy(xs_hbm.at[csrc_v.at[j0]],
                                      rows0_v, sem0).wait()
                pltpu.sync_copy(rows0_v, acc_sh.at[cdst_v.at[j0]], add=True)

                @pl.when(j0 + 2 < ncl)
                def _():
                    pltpu.async_copy(xs_hbm.at[csrc_v.at[j0 + 2]],
                                     rows0_v, sem0)

                pltpu.make_async_copy(xs_hbm.at[csrc_v.at[j1]],
                                      rows1_v, sem1).wait()
                pltpu.sync_copy(rows1_v, acc_sh.at[cdst_v.at[j1]],
                                add=True)

            plsc.subcore_barrier()
            pltpu.sync_copy(acc_sh.at[pl.ds(s * wq, wq)],
                            out_hbm.at[pl.ds(lo + s * wq, wq)])

    return k(xs, src2d, dst2d)


_R = 1024


def _dense_call(agg1, x_pad, dis_c, inv_c, W1, b1r, W2):
    def body(agg_ref, x_ref, dis_ref, inv_ref, w1_ref, b1_ref, w2_ref,
             xs2_ref, p_ref):
        dis = dis_ref[...]
        inv = inv_ref[...]
        z1 = dis * agg_ref[...] + inv * x_ref[...]
        h1 = jnp.maximum(
            jnp.dot(z1.astype(jnp.bfloat16),
                    w1_ref[...].astype(jnp.bfloat16),
                    preferred_element_type=jnp.float32)
            + b1_ref[...], 0.0)
        h2 = jnp.dot(h1.astype(jnp.bfloat16),
                     w2_ref[...].astype(jnp.bfloat16),
                     preferred_element_type=jnp.float32)
        xs2_ref[...] = dis * h2
        p_ref[...] = inv * h2

    return pl.pallas_call(
        body,
        grid=(_NPAD // _R,),
        in_specs=[
            pl.BlockSpec((_R, _IN), lambda i: (i, 0)),
            pl.BlockSpec((_R, _IN), lambda i: (i, 0)),
            pl.BlockSpec((_R, 1), lambda i: (i, 0)),
            pl.BlockSpec((_R, 1), lambda i: (i, 0)),
            pl.BlockSpec((_IN, _HID), lambda i: (0, 0)),
            pl.BlockSpec((1, _HID), lambda i: (0, 0)),
            pl.BlockSpec((_HID, _OUT), lambda i: (0, 0)),
        ],
        out_specs=[
            pl.BlockSpec((_R, _OUT), lambda i: (i, 0)),
            pl.BlockSpec((_R, _OUT), lambda i: (i, 0)),
        ],
        out_shape=[
            jax.ShapeDtypeStruct((_NPAD, _OUT), jnp.float32),
            jax.ShapeDtypeStruct((_NPAD, _OUT), jnp.float32),
        ],
    )(agg1, x_pad, dis_c, inv_c, W1, b1r, W2)


def _final_call(agg2, p, dis_c, b2r):
    def body(agg_ref, p_ref, dis_ref, b2_ref, o_ref):
        o_ref[...] = jnp.maximum(
            dis_ref[...] * agg_ref[...] + p_ref[...] + b2_ref[...], 0.0)

    return pl.pallas_call(
        body,
        grid=(_NPAD // _R,),
        in_specs=[
            pl.BlockSpec((_R, _OUT), lambda i: (i, 0)),
            pl.BlockSpec((_R, _OUT), lambda i: (i, 0)),
            pl.BlockSpec((_R, 1), lambda i: (i, 0)),
            pl.BlockSpec((1, _OUT), lambda i: (0, 0)),
        ],
        out_specs=pl.BlockSpec((_R, _OUT), lambda i: (i, 0)),
        out_shape=jax.ShapeDtypeStruct((_NPAD, _OUT), jnp.float32),
    )(agg2, p, dis_c, b2r)


def kernel(x, edge_index, W1, b1, W2, b2):
    src = edge_index[0]
    dst = edge_index[1]
    n_grp = _EPAD // 5120
    per_grp_pad = 5120 - _E // n_grp
    pad_rows = _N + (jnp.arange(per_grp_pad, dtype=jnp.int32) * 2
                     ) % (_NPAD - _N)
    pad_blk = jnp.broadcast_to(pad_rows, (n_grp, per_grp_pad))
    src_pad = jnp.concatenate(
        [src.reshape(n_grp, -1), pad_blk], axis=1).reshape(-1)
    dst_pad = jnp.concatenate(
        [dst.reshape(n_grp, -1), pad_blk], axis=1).reshape(-1)
    x_pad = jnp.concatenate(
        [x, jnp.zeros((_NPAD - _N, _IN), x.dtype)], axis=0)
    src2d = src_pad.reshape(_EPAD // _CHUNK, _CHUNK)
    dst2d = dst_pad.reshape(_EPAD // _CHUNK, _CHUNK)

    deg_parts = _deg_call(dst2d)
    deg = deg_parts[:_NPAD, 0] + deg_parts[_NPAD:, 0] + 1.0
    dis = jnp.where(jnp.arange(_NPAD) < _N, lax.rsqrt(deg), 0.0)
    inv = 1.0 / deg
    dis_c = dis[:, None]
    inv_c = inv[:, None]

    xs = dis_c * x_pad

    agg1 = _agg_call(xs.reshape(_NPAD, 2, _F), src2d, dst2d
                     ).reshape(_NPAD, _IN)
    xs2, p = _dense_call(agg1, x_pad, dis_c, inv_c,
                         W1, b1.reshape(1, -1), W2)
    agg2 = _agg_call(xs2.reshape(_NPAD, 2, _F), src2d, dst2d
                     ).reshape(_NPAD, _OUT)
    out = _final_call(agg2, p, dis_c, b2.reshape(1, -1))
    return out[:_N]


# speedup vs baseline: 3.8410x; 1.0011x over previous
import functools

import jax
import jax.numpy as jnp
from jax import lax
from jax.experimental import pallas as pl
from jax.experimental.pallas import tpu as pltpu
from jax.experimental.pallas import tpu_sc as plsc

_N = 10000
_E = 160000
_IN = 256
_OUT = 256
_HID = 512

_NPAD = 10240
_EPAD = 163840
_NC = 2
_NS = 16
_F = 128
_CHUNK = 128


def _vmesh():
    return plsc.VectorSubcoreMesh(core_axis_name="c", subcore_axis_name="s")


def _sc_params():
    return pltpu.CompilerParams(needs_layout_passes=False)


def _deg_call(dst2d):
    per_w = _EPAD // (_NC * _NS)
    n_chunks = per_w // _CHUNK
    wb = _NPAD // _NS

    @functools.partial(
        pl.kernel,
        out_type=jax.ShapeDtypeStruct((_NC * _NPAD, _F), jnp.float32),
        mesh=_vmesh(),
        scratch_types=[
            pltpu.VMEM((n_chunks, _CHUNK), jnp.int32),
            pltpu.VMEM((_CHUNK, _F), jnp.float32),
            pltpu.VMEM((_CHUNK, _F), jnp.float32),
            pltpu.VMEM_SHARED((_NPAD, _F), jnp.float32),
            pltpu.SemaphoreType.DMA,
        ],
        compiler_params=_sc_params(),
    )
    def k(dst_hbm, out_hbm, didx_v, ones_v, zbuf_v, hist_sh, sem):
        c = lax.axis_index("c")
        s = lax.axis_index("s")
        w = c * _NS + s

        pltpu.sync_copy(dst_hbm.at[pl.ds(w * n_chunks, n_chunks)], didx_v)

        @pl.loop(0, _CHUNK)
        def _(i):
            for g in range(_F // 16):
                ones_v[i, pl.ds(g * 16, 16)] = jnp.full((16,), 1.0,
                                                        jnp.float32)
                zbuf_v[i, pl.ds(g * 16, 16)] = jnp.zeros((16,), jnp.float32)

        for kk in range(wb // _CHUNK):
            pltpu.sync_copy(zbuf_v,
                            hist_sh.at[pl.ds(s * wb + kk * _CHUNK, _CHUNK)])
        plsc.subcore_barrier()

        @pl.loop(0, n_chunks // 4)
        def _(t):
            for kk in range(4):
                pltpu.async_copy(ones_v, hist_sh.at[didx_v.at[4 * t + kk]],
                                 sem, add=True)
            for kk in range(4):
                pltpu.make_async_copy(ones_v,
                                      hist_sh.at[didx_v.at[4 * t + kk]],
                                      sem).wait()

        plsc.subcore_barrier()
        pltpu.sync_copy(hist_sh.at[pl.ds(s * wb, wb)],
                        out_hbm.at[pl.ds(c * _NPAD + s * wb, wb)])

    return k(dst2d)


_Q = _NPAD // 4
_CAP = 6144


def _agg_call(xs, src2d, dst2d):
    per_s = _EPAD // _NS
    n_chunks = per_s // _CHUNK
    nstage = n_chunks // 2
    wq = _Q // _NS

    @functools.partial(
        pl.kernel,
        out_type=jax.ShapeDtypeStruct((_NPAD, 2, _F), jnp.float32),
        mesh=_vmesh(),
        scratch_types=[
            pltpu.VMEM((nstage, _CHUNK), jnp.int32),
            pltpu.VMEM((nstage, _CHUNK), jnp.int32),
            pltpu.VMEM((_CAP // _CHUNK, _CHUNK), jnp.int32),
            pltpu.VMEM((_CAP // _CHUNK, _CHUNK), jnp.int32),
            pltpu.VMEM((_CHUNK, 2, _F), jnp.float32),
            pltpu.VMEM((_CHUNK, 2, _F), jnp.float32),
            pltpu.VMEM_SHARED((_Q, 2, _F), jnp.float32),
            pltpu.SemaphoreType.DMA,
            pltpu.SemaphoreType.DMA,
        ],
        compiler_params=_sc_params(),
    )
    def k(xs_hbm, src_hbm, dst_hbm, out_hbm,
          sraw_v, draw_v, csrc_v, cdst_v, rows0_v, rows1_v,
          acc_sh, sem0, sem1):
        c = lax.axis_index("c")
        s = lax.axis_index("s")

        for p in range(2):
            q = c * 2 + p
            lo = q * _Q
            hi = lo + _Q

            @pl.loop(0, _CHUNK)
            def _(i):
                for h in range(2):
                    for g in range(_F // 16):
                        rows0_v[i, h, pl.ds(g * 16, 16)] = jnp.zeros(
                            (16,), jnp.float32)

            pltpu.sync_copy(rows0_v, acc_sh.at[pl.ds(s * wq, _CHUNK)])
            pltpu.sync_copy(rows0_v.at[pl.ds(0, wq - _CHUNK)],
                            acc_sh.at[pl.ds(s * wq + _CHUNK, wq - _CHUNK)])

            @pl.loop(0, _CAP // _CHUNK)
            def _(i):
                for g in range(_CHUNK // 16):
                    flat = i * _CHUNK + g * 16 + lax.iota(jnp.int32, 16)
                    csrc_v[i, pl.ds(g * 16, 16)] = (
                        _N + jnp.remainder(flat, _NPAD - _N))
                    cdst_v[i, pl.ds(g * 16, 16)] = jnp.remainder(flat, _Q)

            cnt = jnp.int32(0)
            for phase in range(2):
                pbase = s * n_chunks + phase * nstage
                pltpu.sync_copy(src_hbm.at[pl.ds(pbase, nstage)], sraw_v)
                pltpu.sync_copy(dst_hbm.at[pl.ds(pbase, nstage)], draw_v)

                def rowbody(j, cnt):
                    for g in range(_CHUNK // 16):
                        d = draw_v[j, pl.ds(g * 16, 16)]
                        sv = sraw_v[j, pl.ds(g * 16, 16)]
                        mask = (d >= lo) & (d < hi)
                        mi = mask.astype(jnp.int32)
                        pos = jnp.minimum(cnt + plsc.cumsum(mi) - 1,
                                          _CAP - 1)
                        prow = jnp.right_shift(pos, 7)
                        plane = jnp.bitwise_and(pos, _CHUNK - 1)
                        plsc.store_scatter(cdst_v, [prow, plane], d - lo,
                                           mask=mask)
                        plsc.store_scatter(csrc_v, [prow, plane], sv,
                                           mask=mask)
                        cnt = cnt + lax.reduce_sum(mi, axes=(0,))
                    return cnt

                cnt = lax.fori_loop(0, nstage, rowbody, cnt)

            nu = (cnt + 2 * _CHUNK - 1) // (2 * _CHUNK)
            ncl = 2 * nu
            plsc.subcore_barrier()

            pltpu.async_copy(xs_hbm.at[csrc_v.at[jnp.int32(0)]],
                             rows0_v, sem0)

            @pl.loop(0, nu)
            def _(u):
                j0 = 2 * u
                j1 = j0 + 1

                pltpu.async_copy(xs_hbm.at[csrc_v.at[j1]], rows1_v, sem1)
                pltpu.make_async_copy(xs_hbm.at[csrc_v.at[j0]],
                                      rows0_v, sem0).wait()
                pltpu.sync_copy(rows0_v, acc_sh.at[cdst_v.at[j0]], add=True)

                @pl.when(j0 + 2 < ncl)
                def _():
                    pltpu.async_copy(xs_hbm.at[csrc_v.at[j0 + 2]],
                                     rows0_v, sem0)

                pltpu.make_async_copy(xs_hbm.at[csrc_v.at[j1]],
                                      rows1_v, sem1).wait()
                pltpu.sync_copy(rows1_v, acc_sh.at[cdst_v.at[j1]],
                                add=True)

            plsc.subcore_barrier()
            pltpu.sync_copy(acc_sh.at[pl.ds(s * wq, wq)],
                            out_hbm.at[pl.ds(lo + s * wq, wq)])

    return k(xs, src2d, dst2d)


_R = 1024


def _dense_call(agg1, x_pad, dis_c, inv_c, W1, b1r, W2):
    def body(agg_ref, x_ref, dis_ref, inv_ref, w1_ref, b1_ref, w2_ref,
             xs2_ref, p_ref):
        dis = dis_ref[...]
        inv = inv_ref[...]
        z1 = dis * agg_ref[...] + inv * x_ref[...]
        h1 = jnp.maximum(
            jnp.dot(z1, w1_ref[...], preferred_element_type=jnp.float32)
            + b1_ref[...], 0.0)
        h2 = jnp.dot(h1, w2_ref[...], preferred_element_type=jnp.float32)
        xs2_ref[...] = dis * h2
        p_ref[...] = inv * h2

    return pl.pallas_call(
        body,
        grid=(_NPAD // _R,),
        in_specs=[
            pl.BlockSpec((_R, _IN), lambda i: (i, 0)),
            pl.BlockSpec((_R, _IN), lambda i: (i, 0)),
            pl.BlockSpec((_R, 1), lambda i: (i, 0)),
            pl.BlockSpec((_R, 1), lambda i: (i, 0)),
            pl.BlockSpec((_IN, _HID), lambda i: (0, 0)),
            pl.BlockSpec((1, _HID), lambda i: (0, 0)),
            pl.BlockSpec((_HID, _OUT), lambda i: (0, 0)),
        ],
        out_specs=[
            pl.BlockSpec((_R, _OUT), lambda i: (i, 0)),
            pl.BlockSpec((_R, _OUT), lambda i: (i, 0)),
        ],
        out_shape=[
            jax.ShapeDtypeStruct((_NPAD, _OUT), jnp.float32),
            jax.ShapeDtypeStruct((_NPAD, _OUT), jnp.float32),
        ],
    )(agg1, x_pad, dis_c, inv_c, W1, b1r, W2)


def _final_call(agg2, p, dis_c, b2r):
    def body(agg_ref, p_ref, dis_ref, b2_ref, o_ref):
        o_ref[...] = jnp.maximum(
            dis_ref[...] * agg_ref[...] + p_ref[...] + b2_ref[...], 0.0)

    return pl.pallas_call(
        body,
        grid=(_NPAD // _R,),
        in_specs=[
            pl.BlockSpec((_R, _OUT), lambda i: (i, 0)),
            pl.BlockSpec((_R, _OUT), lambda i: (i, 0)),
            pl.BlockSpec((_R, 1), lambda i: (i, 0)),
            pl.BlockSpec((1, _OUT), lambda i: (0, 0)),
        ],
        out_specs=pl.BlockSpec((_R, _OUT), lambda i: (i, 0)),
        out_shape=jax.ShapeDtypeStruct((_NPAD, _OUT), jnp.float32),
    )(agg2, p, dis_c, b2r)


def kernel(x, edge_index, W1, b1, W2, b2):
    src = edge_index[0]
    dst = edge_index[1]
    n_grp = _EPAD // 5120
    per_grp_pad = 5120 - _E // n_grp
    pad_rows = _N + (jnp.arange(per_grp_pad, dtype=jnp.int32) * 2
                     ) % (_NPAD - _N)
    pad_blk = jnp.broadcast_to(pad_rows, (n_grp, per_grp_pad))
    src_pad = jnp.concatenate(
        [src.reshape(n_grp, -1), pad_blk], axis=1).reshape(-1)
    dst_pad = jnp.concatenate(
        [dst.reshape(n_grp, -1), pad_blk], axis=1).reshape(-1)
    x_pad = jnp.concatenate(
        [x, jnp.zeros((_NPAD - _N, _IN), x.dtype)], axis=0)
    src2d = src_pad.reshape(_EPAD // _CHUNK, _CHUNK)
    dst2d = dst_pad.reshape(_EPAD // _CHUNK, _CHUNK)

    deg_parts = _deg_call(dst2d)
    deg = deg_parts[:_NPAD, 0] + deg_parts[_NPAD:, 0] + 1.0
    dis = jnp.where(jnp.arange(_NPAD) < _N, lax.rsqrt(deg), 0.0)
    inv = 1.0 / deg
    dis_c = dis[:, None]
    inv_c = inv[:, None]

    xs = dis_c * x_pad

    agg1 = _agg_call(xs.reshape(_NPAD, 2, _F), src2d, dst2d
                     ).reshape(_NPAD, _IN)
    xs2, p = _dense_call(agg1, x_pad, dis_c, inv_c,
                         W1, b1.reshape(1, -1), W2)
    agg2 = _agg_call(xs2.reshape(_NPAD, 2, _F), src2d, dst2d
                     ).reshape(_NPAD, _OUT)
    out = _final_call(agg2, p, dis_c, b2.reshape(1, -1))
    return out[:_N]
